# baseline (speedup 1.0000x reference)
import functools

import jax
import jax.numpy as jnp
from jax import lax
from jax.experimental import pallas as pl
from jax.experimental.pallas import tpu as pltpu
from jax.experimental.pallas import tpu_sc as plsc

_SU, _SL, _TU, _TL = 1000.0, 0.0, 500.0, 0.0
_BB = 2
_DPAD = 128
_CHUNKS = 2


def _pick_rows_per_worker(n_rows):
    for b_per_w in (128, 64, 32, 16, 8):
        if n_rows % b_per_w == 0 and n_rows // b_per_w <= 32:
            return b_per_w
    return None


def _sc_gather(table, idx, n_rows):
    info = plsc.get_sparse_core_info()
    b_per_w = _pick_rows_per_worker(n_rows)
    n_active = n_rows // b_per_w
    d = table.shape[1]
    mesh = plsc.VectorSubcoreMesh(core_axis_name="c", subcore_axis_name="s")

    @functools.partial(
        pl.kernel, mesh=mesh,
        out_type=jax.ShapeDtypeStruct((n_rows, d), jnp.float32),
        scratch_types=[
            pltpu.VMEM((b_per_w,), jnp.int32),
            pltpu.VMEM((b_per_w, d), jnp.float32),
            pltpu.SemaphoreType.DMA,
        ],
    )
    def k(table_hbm, idx_hbm, out_hbm, idx_v, rows_v, sem):
        wid = lax.axis_index("s") * info.num_cores + lax.axis_index("c")

        @pl.when(wid < n_active)
        def _():
            base = wid * b_per_w
            pltpu.sync_copy(idx_hbm.at[pl.ds(base, b_per_w)], idx_v)
            pltpu.async_copy(table_hbm.at[idx_v], rows_v, sem).wait()
            pltpu.sync_copy(rows_v, out_hbm.at[pl.ds(base, b_per_w)])

    return k(table, idx)


def _expand_kernel(lmax_ref,
                   ds_ref, tlen_ref, su_ref, sl_ref, tu_ref, tl_ref,
                   *rest):
    out_ref = rest[-1]
    bb, m_sz, l, e = out_ref.shape
    rows = bb * m_sz

    lmax = lmax_ref[0]
    m_pp = jax.lax.broadcasted_iota(jnp.int32, (rows, 1), 0) % m_sz
    v2 = m_pp < tlen_ref[0]
    col_ok = jax.lax.broadcasted_iota(jnp.int32, (rows, l), 1) < lmax
    ds = jnp.where(v2 & col_ok, ds_ref[0, :, :l], 0.0)

    a_tab = (sl_ref[...] * _SU - su_ref[...] * _SL) * (1.0 / (_SU - _SL)) + \
            (tl_ref[...] * _TU - tu_ref[...] * _TL) * (1.0 / (_TU - _TL))
    b_tab = (su_ref[...] - sl_ref[...]) * (1.0 / (_SU - _SL)) + \
            (tu_ref[...] - tl_ref[...]) * (1.0 / (_TU - _TL))
    a_v = jnp.where(v2, a_tab[1:2, :], a_tab[0:1, :])
    b_v = jnp.where(v2, b_tab[1:2, :], b_tab[0:1, :])

    val = a_v[:, None, :] + b_v[:, None, :] * ds[:, :, None]
    for t in range(bb):
        out_ref[t] = val[t * m_sz:(t + 1) * m_sz]


def _tc_expand_chunk(chunk, n_chunks, shape, prev_out, lmax_arr, ds3, tlen_pp,
                     emb_su, emb_sl, emb_tu, emb_tl):
    b_sz, m_sz, l_sz, e_sz = shape
    bb = _BB if (b_sz // n_chunks) % _BB == 0 else 1
    steps = b_sz // n_chunks // bb
    rows = bb * m_sz
    step0 = chunk * steps
    full = lambda s, *refs: (0, 0)

    in_specs = [
        pl.BlockSpec((1, rows, _DPAD), lambda s, *refs: (s, 0, 0)),
        pl.BlockSpec((1, rows, 1), lambda s, *refs: (s, 0, 0)),
        pl.BlockSpec((2, e_sz), full),
        pl.BlockSpec((2, e_sz), full),
        pl.BlockSpec((2, e_sz), full),
        pl.BlockSpec((2, e_sz), full),
    ]
    args = [lmax_arr, ds3, tlen_pp, emb_su, emb_sl, emb_tu, emb_tl]
    aliases = {}
    if prev_out is not None:
        in_specs.append(pl.BlockSpec(memory_space=pl.ANY))
        args.append(prev_out)
        aliases = {7: 0}

    return pl.pallas_call(
        _expand_kernel,
        grid_spec=pltpu.PrefetchScalarGridSpec(
            num_scalar_prefetch=1,
            grid=(steps,),
            in_specs=in_specs,
            out_specs=pl.BlockSpec((bb, m_sz, l_sz, e_sz),
                                   lambda s, *refs: (step0 + s, 0, 0, 0)),
        ),
        out_shape=jax.ShapeDtypeStruct((b_sz, m_sz, l_sz, e_sz), jnp.float32),
        input_output_aliases=aliases,
    )(*args)


def kernel(traj_loc, mat2, vec, traj_len, l_max, emb_su, emb_sl, emb_tu, emb_tl):
    del vec
    b_sz, m_sz = traj_loc.shape
    n_loc, l_sz = mat2.shape
    e_sz = emb_su.shape[1]
    n_chunks = _CHUNKS if b_sz % (_CHUNKS * _BB) == 0 else 1
    b_c = b_sz // n_chunks
    bb = _BB if b_c % _BB == 0 else 1
    rows = bb * m_sz
    pairs_c = b_c * m_sz

    table = jnp.pad(mat2, ((1, 0), (0, _DPAD - l_sz)))
    idx = traj_loc.astype(jnp.int32).reshape(n_chunks, pairs_c)
    tlen = jnp.repeat(traj_len.astype(jnp.int32), m_sz
                      ).reshape(n_chunks, pairs_c // rows, rows, 1)
    lmax_arr = jnp.asarray(l_max, jnp.int32).reshape(1)

    shape = (b_sz, m_sz, l_sz, e_sz)
    out = None
    for c in range(n_chunks):
        ds_rows = _sc_gather(table, idx[c], pairs_c)
        ds3 = ds_rows.reshape(pairs_c // rows, rows, _DPAD)
        out = _tc_expand_chunk(c, n_chunks, shape, out, lmax_arr, ds3,
                               tlen[c], emb_su, emb_sl, emb_tu, emb_tl)
    return out


# speedup vs baseline: 1.0285x; 1.0285x over previous
import functools

import jax
import jax.numpy as jnp
from jax import lax
from jax.experimental import pallas as pl
from jax.experimental.pallas import tpu as pltpu
from jax.experimental.pallas import tpu_sc as plsc

_SU, _SL, _TU, _TL = 1000.0, 0.0, 500.0, 0.0
_BB = 2
_DPAD = 128


def _pick_rows_per_worker(n_rows):
    for b_per_w in (128, 64, 32, 16, 8):
        if n_rows % b_per_w == 0 and n_rows // b_per_w <= 32:
            return b_per_w
    return None


def _sc_gather(table, idx, n_rows):
    info = plsc.get_sparse_core_info()
    b_per_w = _pick_rows_per_worker(n_rows)
    n_active = n_rows // b_per_w
    d = table.shape[1]
    mesh = plsc.VectorSubcoreMesh(core_axis_name="c", subcore_axis_name="s")

    @functools.partial(
        pl.kernel, mesh=mesh,
        out_type=jax.ShapeDtypeStruct((n_rows, d), jnp.float32),
        scratch_types=[
            pltpu.VMEM((b_per_w,), jnp.int32),
            pltpu.VMEM((b_per_w, d), jnp.float32),
            pltpu.SemaphoreType.DMA,
        ],
    )
    def k(table_hbm, idx_hbm, out_hbm, idx_v, rows_v, sem):
        wid = lax.axis_index("s") * info.num_cores + lax.axis_index("c")

        @pl.when(wid < n_active)
        def _():
            base = wid * b_per_w
            pltpu.sync_copy(idx_hbm.at[pl.ds(base, b_per_w)], idx_v)
            pltpu.async_copy(table_hbm.at[idx_v], rows_v, sem).wait()
            pltpu.sync_copy(rows_v, out_hbm.at[pl.ds(base, b_per_w)])

    return k(table, idx)


def _expand_kernel(lmax_ref,
                   ds_ref, tlen_ref, su_ref, sl_ref, tu_ref, tl_ref,
                   out_ref):
    bb, m_sz, l, e = out_ref.shape
    rows = bb * m_sz

    lmax = lmax_ref[0]
    m_pp = jax.lax.broadcasted_iota(jnp.int32, (rows, 1), 0) % m_sz
    v2 = m_pp < tlen_ref[0]
    col_ok = jax.lax.broadcasted_iota(jnp.int32, (rows, l), 1) < lmax
    ds = jnp.where(v2 & col_ok, ds_ref[0, :, :l], 0.0)

    a_tab = (sl_ref[...] * _SU - su_ref[...] * _SL) * (1.0 / (_SU - _SL)) + \
            (tl_ref[...] * _TU - tu_ref[...] * _TL) * (1.0 / (_TU - _TL))
    b_tab = (su_ref[...] - sl_ref[...]) * (1.0 / (_SU - _SL)) + \
            (tu_ref[...] - tl_ref[...]) * (1.0 / (_TU - _TL))
    a_v = jnp.where(v2, a_tab[1:2, :], a_tab[0:1, :])
    b_v = jnp.where(v2, b_tab[1:2, :], b_tab[0:1, :])

    val = a_v[:, None, :] + b_v[:, None, :] * ds[:, :, None]
    for t in range(bb):
        out_ref[t] = val[t * m_sz:(t + 1) * m_sz]


def kernel(traj_loc, mat2, vec, traj_len, l_max, emb_su, emb_sl, emb_tu, emb_tl):
    del vec
    b_sz, m_sz = traj_loc.shape
    n_loc, l_sz = mat2.shape
    e_sz = emb_su.shape[1]
    bb = _BB if b_sz % _BB == 0 else 1
    grid = (b_sz // bb,)
    rows = bb * m_sz

    table = jnp.pad(mat2, ((1, 0), (0, _DPAD - l_sz)))
    n_pairs = b_sz * m_sz
    idx = traj_loc.astype(jnp.int32).reshape(-1)
    ds_rows = _sc_gather(table, idx, n_pairs)
    ds3 = ds_rows.reshape(n_pairs // rows, rows, _DPAD)

    tlen_pp = jnp.repeat(traj_len.astype(jnp.int32), m_sz
                         ).reshape(n_pairs // rows, rows, 1)

    lmax_arr = jnp.asarray(l_max, jnp.int32).reshape(1)
    full = lambda s, *refs: (0, 0)

    out = pl.pallas_call(
        _expand_kernel,
        grid_spec=pltpu.PrefetchScalarGridSpec(
            num_scalar_prefetch=1,
            grid=grid,
            in_specs=[
                pl.BlockSpec((1, rows, _DPAD), lambda s, *refs: (s, 0, 0)),
                pl.BlockSpec((1, rows, 1), lambda s, *refs: (s, 0, 0)),
                pl.BlockSpec((2, e_sz), full),
                pl.BlockSpec((2, e_sz), full),
                pl.BlockSpec((2, e_sz), full),
                pl.BlockSpec((2, e_sz), full),
            ],
            out_specs=pl.BlockSpec((bb, m_sz, l_sz, e_sz),
                                   lambda s, *refs: (s, 0, 0, 0)),
        ),
        out_shape=jax.ShapeDtypeStruct((b_sz, m_sz, l_sz, e_sz), jnp.float32),
    )(lmax_arr, ds3, tlen_pp, emb_su, emb_sl, emb_tu, emb_tl)
    return out


# speedup vs baseline: 1.0330x; 1.0045x over previous
import functools

import jax
import jax.numpy as jnp
from jax import lax
from jax.experimental import pallas as pl
from jax.experimental.pallas import tpu as pltpu
from jax.experimental.pallas import tpu_sc as plsc

_SU, _SL, _TU, _TL = 1000.0, 0.0, 500.0, 0.0
_BB = 2
_DPAD = 128


def _pick_rows_per_worker(n_rows):
    for b_per_w in (128, 64, 32, 16, 8):
        if n_rows % b_per_w == 0 and n_rows // b_per_w <= 32:
            return b_per_w
    return None


def _sc_gather(table, idx, n_rows):
    info = plsc.get_sparse_core_info()
    b_per_w = _pick_rows_per_worker(n_rows)
    n_active = n_rows // b_per_w
    d = table.shape[1]
    mesh = plsc.VectorSubcoreMesh(core_axis_name="c", subcore_axis_name="s")

    @functools.partial(
        pl.kernel, mesh=mesh,
        out_type=jax.ShapeDtypeStruct((n_rows, d), jnp.float32),
        scratch_types=[
            pltpu.VMEM((b_per_w,), jnp.int32),
            pltpu.VMEM((b_per_w, d), jnp.float32),
            pltpu.SemaphoreType.DMA,
        ],
    )
    def k(table_hbm, idx_hbm, out_hbm, idx_v, rows_v, sem):
        wid = lax.axis_index("s") * info.num_cores + lax.axis_index("c")

        @pl.when(wid < n_active)
        def _():
            base = wid * b_per_w
            pltpu.sync_copy(idx_hbm.at[pl.ds(base, b_per_w)], idx_v)
            pltpu.async_copy(table_hbm.at[idx_v], rows_v, sem).wait()
            pltpu.sync_copy(rows_v, out_hbm.at[pl.ds(base, b_per_w)])

    return k(table, idx)


def _expand_kernel(len_ref, lmax_ref,
                   ds_ref, su_ref, sl_ref, tu_ref, tl_ref,
                   out_ref):
    s = pl.program_id(0)
    bb, m_sz, l, e = out_ref.shape
    rows = bb * m_sz

    lmax = lmax_ref[0]
    r_pp = jax.lax.broadcasted_iota(jnp.int32, (rows, 1), 0)
    m_pp = r_pp % m_sz
    tlen = len_ref[s * bb]
    for t in range(1, bb):
        tlen = jnp.where(r_pp >= t * m_sz, len_ref[s * bb + t], tlen)
    v2 = m_pp < tlen
    col_ok = jax.lax.broadcasted_iota(jnp.int32, (rows, l), 1) < lmax
    ds = jnp.where(v2 & col_ok, ds_ref[0, :, :l], 0.0)

    a_tab = (sl_ref[...] * _SU - su_ref[...] * _SL) * (1.0 / (_SU - _SL)) + \
            (tl_ref[...] * _TU - tu_ref[...] * _TL) * (1.0 / (_TU - _TL))
    b_tab = (su_ref[...] - sl_ref[...]) * (1.0 / (_SU - _SL)) + \
            (tu_ref[...] - tl_ref[...]) * (1.0 / (_TU - _TL))
    a_v = jnp.where(v2, a_tab[1:2, :], a_tab[0:1, :])
    b_v = jnp.where(v2, b_tab[1:2, :], b_tab[0:1, :])

    val = a_v[:, None, :] + b_v[:, None, :] * ds[:, :, None]
    for t in range(bb):
        out_ref[t] = val[t * m_sz:(t + 1) * m_sz]


def kernel(traj_loc, mat2, vec, traj_len, l_max, emb_su, emb_sl, emb_tu, emb_tl):
    del vec
    b_sz, m_sz = traj_loc.shape
    n_loc, l_sz = mat2.shape
    e_sz = emb_su.shape[1]
    bb = _BB if b_sz % _BB == 0 else 1
    grid = (b_sz // bb,)
    rows = bb * m_sz

    table = jnp.pad(mat2, ((1, 0), (0, _DPAD - l_sz)))
    n_pairs = b_sz * m_sz
    idx = traj_loc.astype(jnp.int32).reshape(-1)
    ds_rows = _sc_gather(table, idx, n_pairs)
    ds3 = ds_rows.reshape(n_pairs // rows, rows, _DPAD)

    lmax_arr = jnp.asarray(l_max, jnp.int32).reshape(1)
    full = lambda s, *refs: (0, 0)

    out = pl.pallas_call(
        _expand_kernel,
        grid_spec=pltpu.PrefetchScalarGridSpec(
            num_scalar_prefetch=2,
            grid=grid,
            in_specs=[
                pl.BlockSpec((1, rows, _DPAD), lambda s, *refs: (s, 0, 0)),
                pl.BlockSpec((2, e_sz), full),
                pl.BlockSpec((2, e_sz), full),
                pl.BlockSpec((2, e_sz), full),
                pl.BlockSpec((2, e_sz), full),
            ],
            out_specs=pl.BlockSpec((bb, m_sz, l_sz, e_sz),
                                   lambda s, *refs: (s, 0, 0, 0)),
        ),
        out_shape=jax.ShapeDtypeStruct((b_sz, m_sz, l_sz, e_sz), jnp.float32),
    )(traj_len.astype(jnp.int32), lmax_arr,
      ds3, emb_su, emb_sl, emb_tu, emb_tl)
    return out


# speedup vs baseline: 1.0334x; 1.0003x over previous
import functools

import jax
import jax.numpy as jnp
from jax import lax
from jax.experimental import pallas as pl
from jax.experimental.pallas import tpu as pltpu
from jax.experimental.pallas import tpu_sc as plsc

_SU, _SL, _TU, _TL = 1000.0, 0.0, 500.0, 0.0
_BB = 2
_DPAD = 128


def _pick_rows_per_worker(n_rows):
    for b_per_w in (128, 64, 32, 16, 8):
        if n_rows % b_per_w == 0 and n_rows // b_per_w <= 32:
            return b_per_w
    return None


def _sc_gather(table, idx, n_rows):
    info = plsc.get_sparse_core_info()
    b_per_w = _pick_rows_per_worker(n_rows)
    n_active = n_rows // b_per_w
    d = table.shape[1]
    mesh = plsc.VectorSubcoreMesh(core_axis_name="c", subcore_axis_name="s")

    h = b_per_w // 2

    @functools.partial(
        pl.kernel, mesh=mesh,
        out_type=jax.ShapeDtypeStruct((n_rows, d), jnp.float32),
        scratch_types=[
            pltpu.VMEM((b_per_w,), jnp.int32),
            pltpu.VMEM((2, h, d), jnp.float32),
            pltpu.SemaphoreType.DMA,
            pltpu.SemaphoreType.DMA,
            pltpu.SemaphoreType.DMA,
        ],
    )
    def k(table_hbm, idx_hbm, out_hbm, idx_v, rows_v, sem0, sem1, sem2):
        wid = lax.axis_index("s") * info.num_cores + lax.axis_index("c")

        @pl.when(wid < n_active)
        def _():
            base = wid * b_per_w
            pltpu.sync_copy(idx_hbm.at[pl.ds(base, b_per_w)], idx_v)
            g0 = pltpu.async_copy(table_hbm.at[idx_v.at[pl.ds(0, h)]],
                                  rows_v.at[0], sem0)
            g1 = pltpu.async_copy(table_hbm.at[idx_v.at[pl.ds(h, h)]],
                                  rows_v.at[1], sem1)
            g0.wait()
            w0 = pltpu.async_copy(rows_v.at[0],
                                  out_hbm.at[pl.ds(base, h)], sem2)
            g1.wait()
            pltpu.sync_copy(rows_v.at[1], out_hbm.at[pl.ds(base + h, h)])
            w0.wait()

    return k(table, idx)


def _expand_kernel(len_ref, lmax_ref,
                   ds_ref, su_ref, sl_ref, tu_ref, tl_ref,
                   out_ref):
    s = pl.program_id(0)
    bb, m_sz, l, e = out_ref.shape
    rows = bb * m_sz

    lmax = lmax_ref[0]
    r_pp = jax.lax.broadcasted_iota(jnp.int32, (rows, 1), 0)
    m_pp = r_pp % m_sz
    tlen = len_ref[s * bb]
    for t in range(1, bb):
        tlen = jnp.where(r_pp >= t * m_sz, len_ref[s * bb + t], tlen)
    v2 = m_pp < tlen
    col_ok = jax.lax.broadcasted_iota(jnp.int32, (rows, l), 1) < lmax
    ds = jnp.where(v2 & col_ok, ds_ref[0, :, :l], 0.0)

    a_tab = (sl_ref[...] * _SU - su_ref[...] * _SL) * (1.0 / (_SU - _SL)) + \
            (tl_ref[...] * _TU - tu_ref[...] * _TL) * (1.0 / (_TU - _TL))
    b_tab = (su_ref[...] - sl_ref[...]) * (1.0 / (_SU - _SL)) + \
            (tu_ref[...] - tl_ref[...]) * (1.0 / (_TU - _TL))
    a_v = jnp.where(v2, a_tab[1:2, :], a_tab[0:1, :])
    b_v = jnp.where(v2, b_tab[1:2, :], b_tab[0:1, :])

    val = a_v[:, None, :] + b_v[:, None, :] * ds[:, :, None]
    for t in range(bb):
        out_ref[t] = val[t * m_sz:(t + 1) * m_sz]


def kernel(traj_loc, mat2, vec, traj_len, l_max, emb_su, emb_sl, emb_tu, emb_tl):
    del vec
    b_sz, m_sz = traj_loc.shape
    n_loc, l_sz = mat2.shape
    e_sz = emb_su.shape[1]
    bb = _BB if b_sz % _BB == 0 else 1
    grid = (b_sz // bb,)
    rows = bb * m_sz

    table = jnp.pad(mat2, ((1, 0), (0, _DPAD - l_sz)))
    n_pairs = b_sz * m_sz
    idx = traj_loc.astype(jnp.int32).reshape(-1)
    ds_rows = _sc_gather(table, idx, n_pairs)
    ds3 = ds_rows.reshape(n_pairs // rows, rows, _DPAD)

    lmax_arr = jnp.asarray(l_max, jnp.int32).reshape(1)
    full = lambda s, *refs: (0, 0)

    out = pl.pallas_call(
        _expand_kernel,
        grid_spec=pltpu.PrefetchScalarGridSpec(
            num_scalar_prefetch=2,
            grid=grid,
            in_specs=[
                pl.BlockSpec((1, rows, _DPAD), lambda s, *refs: (s, 0, 0)),
                pl.BlockSpec((2, e_sz), full),
                pl.BlockSpec((2, e_sz), full),
                pl.BlockSpec((2, e_sz), full),
                pl.BlockSpec((2, e_sz), full),
            ],
            out_specs=pl.BlockSpec((bb, m_sz, l_sz, e_sz),
                                   lambda s, *refs: (s, 0, 0, 0)),
        ),
        out_shape=jax.ShapeDtypeStruct((b_sz, m_sz, l_sz, e_sz), jnp.float32),
    )(traj_len.astype(jnp.int32), lmax_arr,
      ds3, emb_su, emb_sl, emb_tu, emb_tl)
    return out
